# baseline (speedup 1.0000x reference)
import functools

import jax
import jax.numpy as jnp
from jax import lax
from jax.experimental import pallas as pl
from jax.experimental.pallas import tpu as pltpu
from jax.experimental.pallas import tpu_sc as plsc

NC, NS, L = 2, 16, 16
NW = NC * NS
BA = 64
BD = 128

_F32 = jnp.float32
_HI = jax.lax.Precision.HIGHEST
_GDN = lax.GatherDimensionNumbers(offset_dims=(), collapsed_slice_dims=(0,),
                                  start_index_map=(0,))


def _rup(a, b):
    return (a + b - 1) // b * b


def _lrelu(x):
    return jnp.where(x >= 0, x, 0.2 * x)


def _dot3(a, b):
    ah = a.astype(jnp.bfloat16)
    al = (a - ah.astype(_F32)).astype(jnp.bfloat16)
    bh = b.astype(jnp.bfloat16)
    bl = (b - bh.astype(_F32)).astype(jnp.bfloat16)
    d = jnp.dot(ah, bh, preferred_element_type=_F32)
    d = d + jnp.dot(ah, bl, preferred_element_type=_F32)
    d = d + jnp.dot(al, bh, preferred_element_type=_F32)
    return d


def _tc_encode1(xh, xl, W1h, W1l, a1w, Nt, H, Dh, R=512):
    Din = xh.shape[1]

    def body(xh_ref, xl_ref, wh_ref, wl_ref, aw_ref, h_ref, a_ref):
        xhb, xlb = xh_ref[...], xl_ref[...]
        whb, wlb = wh_ref[...], wl_ref[...]
        hb = jnp.dot(xhb, whb, preferred_element_type=_F32)
        hb = hb + jnp.dot(xhb, wlb, preferred_element_type=_F32)
        hb = hb + jnp.dot(xlb, whb, preferred_element_type=_F32)
        for hh in range(H):
            sl = hb[:, hh * Dh:(hh + 1) * Dh]
            h_ref[hh] = sl
            a_ref[hh] = jnp.dot(sl, aw_ref[hh], preferred_element_type=_F32,
                                precision=_HI)

    return pl.pallas_call(
        body,
        grid=(Nt // R,),
        in_specs=[
            pl.BlockSpec((R, Din), lambda i: (i, 0)),
            pl.BlockSpec((R, Din), lambda i: (i, 0)),
            pl.BlockSpec((Din, H * Dh), lambda i: (0, 0)),
            pl.BlockSpec((Din, H * Dh), lambda i: (0, 0)),
            pl.BlockSpec((H, Dh, 2), lambda i: (0, 0, 0)),
        ],
        out_specs=[
            pl.BlockSpec((H, R, Dh), lambda i: (0, i, 0)),
            pl.BlockSpec((H, R, 2), lambda i: (0, i, 0)),
        ],
        out_shape=[
            jax.ShapeDtypeStruct((H, Nt, Dh), _F32),
            jax.ShapeDtypeStruct((H, Nt, 2), _F32),
        ],
    )(xh, xl, W1h, W1l, a1w)


def _tc_combine1(acc1, den1t, h1_3, a1m, W2h, W2l, b1r, a2w, Nt, H, Dh, R=512):
    def body(acc_ref, den_ref, h1_ref, a1_ref, wh_ref, wl_ref, b1_ref, aw_ref,
             h2_ref, a2_ref):
        es = []
        for h in range(H):
            a1b = a1_ref[h]
            s = a1b[:, 0:1] + a1b[:, 1:2]
            wself = jnp.exp(_lrelu(s))
            num = acc_ref[h] + acc_ref[H + h] + wself * h1_ref[h]
            den = jnp.sum(den_ref[h], axis=1, keepdims=True) + wself
            o = num / (den + 1e-16) + b1_ref[h]
            es.append(jnp.where(o > 0, o, jnp.exp(o) - 1.0))
        e = jnp.concatenate(es, axis=1)
        eh = e.astype(jnp.bfloat16)
        el = (e - eh.astype(_F32)).astype(jnp.bfloat16)
        whb, wlb = wh_ref[...], wl_ref[...]
        h2 = jnp.dot(eh, whb, preferred_element_type=_F32)
        h2 = h2 + jnp.dot(eh, wlb, preferred_element_type=_F32)
        h2 = h2 + jnp.dot(el, whb, preferred_element_type=_F32)
        h2_ref[...] = h2
        a2_ref[...] = jnp.dot(h2, aw_ref[...], preferred_element_type=_F32,
                              precision=_HI)

    return pl.pallas_call(
        body,
        grid=(Nt // R,),
        in_specs=[
            pl.BlockSpec((NC * H, R, Dh), lambda i: (0, i, 0)),
            pl.BlockSpec((H, R, NW), lambda i: (0, i, 0)),
            pl.BlockSpec((H, R, Dh), lambda i: (0, i, 0)),
            pl.BlockSpec((H, R, 2), lambda i: (0, i, 0)),
            pl.BlockSpec((H * Dh, Dh), lambda i: (0, 0)),
            pl.BlockSpec((H * Dh, Dh), lambda i: (0, 0)),
            pl.BlockSpec((H, 1, Dh), lambda i: (0, 0, 0)),
            pl.BlockSpec((Dh, 2), lambda i: (0, 0)),
        ],
        out_specs=[
            pl.BlockSpec((R, Dh), lambda i: (i, 0)),
            pl.BlockSpec((R, 2), lambda i: (i, 0)),
        ],
        out_shape=[
            jax.ShapeDtypeStruct((Nt, Dh), _F32),
            jax.ShapeDtypeStruct((Nt, 2), _F32),
        ],
    )(acc1, den1t, h1_3, a1m, W2h, W2l, b1r, a2w)


def _tc_combine2(acc2, den2t, h2, a2m, W5h, W5l, b2r, bp1r, Nt, Dh, R=512):
    def body(acc_ref, den_ref, h2_ref, a2_ref, wh_ref, wl_ref, b2_ref,
             bp1_ref, u_ref, v_ref):
        a2b = a2_ref[...]
        s = a2b[:, 0:1] + a2b[:, 1:2]
        wself = jnp.exp(_lrelu(s))
        num = acc_ref[0] + acc_ref[1] + wself * h2_ref[...]
        den = jnp.sum(den_ref[...], axis=1, keepdims=True) + wself
        z = num / (den + 1e-16) + b2_ref[...]
        zh = z.astype(jnp.bfloat16)
        zl = (z - zh.astype(_F32)).astype(jnp.bfloat16)
        whb, wlb = wh_ref[...], wl_ref[...]
        uv = jnp.dot(zh, whb, preferred_element_type=_F32)
        uv = uv + jnp.dot(zh, wlb, preferred_element_type=_F32)
        uv = uv + jnp.dot(zl, whb, preferred_element_type=_F32)
        u_ref[...] = uv[:, :Dh] + bp1_ref[...]
        v_ref[...] = uv[:, Dh:]

    return pl.pallas_call(
        body,
        grid=(Nt // R,),
        in_specs=[
            pl.BlockSpec((NC, R, Dh), lambda i: (0, i, 0)),
            pl.BlockSpec((R, NW), lambda i: (i, 0)),
            pl.BlockSpec((R, Dh), lambda i: (i, 0)),
            pl.BlockSpec((R, 2), lambda i: (i, 0)),
            pl.BlockSpec((Dh, 2 * Dh), lambda i: (0, 0)),
            pl.BlockSpec((Dh, 2 * Dh), lambda i: (0, 0)),
            pl.BlockSpec((1, Dh), lambda i: (0, 0)),
            pl.BlockSpec((1, Dh), lambda i: (0, 0)),
        ],
        out_specs=[
            pl.BlockSpec((R, Dh), lambda i: (i, 0)),
            pl.BlockSpec((R, Dh), lambda i: (i, 0)),
        ],
        out_shape=[
            jax.ShapeDtypeStruct((Nt, Dh), _F32),
            jax.ShapeDtypeStruct((Nt, Dh), _F32),
        ],
    )(acc2, den2t, h2, a2m, W5h, W5l, b2r, bp1r)


def _tc_finish(P2, S, bb, R=1024):
    M = P2.shape[0]

    def body(p_ref, s_ref, b_ref, o_ref):
        o_ref[...] = jnp.dot(p_ref[...], s_ref[...], preferred_element_type=_F32,
                             precision=_HI) + b_ref[...]

    return pl.pallas_call(
        body,
        grid=(M // R,),
        in_specs=[
            pl.BlockSpec((R, 128), lambda i: (i, 0)),
            pl.BlockSpec((128, 8), lambda i: (0, 0)),
            pl.BlockSpec((1, 8), lambda i: (0, 0)),
        ],
        out_specs=pl.BlockSpec((R, 8), lambda i: (i, 0)),
        out_shape=jax.ShapeDtypeStruct((M, 8), _F32),
    )(P2, S, bb)


def _sc_agg(h_flat, atab, srcp, dstp, H, Nt, Dh, Ep):
    B = BA
    epw = Ep // NW
    nb = epw // B
    rpt = Nt // NS
    nzc = rpt // B
    mesh = plsc.VectorSubcoreMesh(core_axis_name="c", subcore_axis_name="s",
                                  num_cores=NC, num_subcores=NS)

    @functools.partial(
        pl.kernel, mesh=mesh,
        compiler_params=pltpu.CompilerParams(needs_layout_passes=False,
                                             internal_scratch_in_bytes=2048),
        out_type=[
            jax.ShapeDtypeStruct((NC * H * Nt, Dh), _F32),
            jax.ShapeDtypeStruct((NW * H * Nt,), _F32),
        ],
        scratch_types=[
            pltpu.VMEM((2 * Nt,), _F32),
            pltpu.VMEM((Nt,), _F32),
            pltpu.VMEM((2, B, Dh), _F32),
            pltpu.VMEM((2, B), jnp.int32),
            pltpu.VMEM((2, B), jnp.int32),
            pltpu.VMEM((2, B), jnp.int32),
            pltpu.VMEM((2, B), jnp.int32),
            pltpu.VMEM((2, B), _F32),
            pltpu.VMEM_SHARED((Nt, Dh), _F32),
            pltpu.SemaphoreType.DMA,
            pltpu.SemaphoreType.DMA,
            pltpu.SemaphoreType.DMA,
            pltpu.SemaphoreType.DMA,
            pltpu.SemaphoreType.DMA,
            pltpu.SemaphoreType.DMA,
        ])
    def k(h_hbm, atab_hbm, src_hbm, dst_hbm, acc_hbm, den_hbm,
          a_vm, den_vm, rows2, srcb2, dstb2, srcadj2, dstsc2, wbuf2,
          acc_sh, isem0, isem1, gsem0, gsem1, ssem0, ssem1):
        c = lax.axis_index("c")
        s = lax.axis_index("s")
        wid = s * NC + c
        zv = jnp.zeros((L,), _F32)
        m0 = lax.iota(jnp.int32, L) == 0
        isem = (isem0, isem1)
        gsem = (gsem0, gsem1)
        ssem = (ssem0, ssem1)
        base = wid * epw

        def issue_idx(g, p):
            eo = base + g * B
            pltpu.async_copy(src_hbm.at[pl.ds(eo, B)], srcb2.at[p], isem[p])
            pltpu.async_copy(dst_hbm.at[pl.ds(eo, B)], dstb2.at[p], isem[p])

        def wait_idx(p):
            pltpu.make_async_copy(src_hbm.at[pl.ds(0, B)], srcb2.at[p],
                                  isem[p]).wait()
            pltpu.make_async_copy(dst_hbm.at[pl.ds(0, B)], dstb2.at[p],
                                  isem[p]).wait()

        def wait_scat(p):
            pltpu.make_async_copy(rows2.at[p], acc_sh.at[pl.ds(0, B)],
                                  ssem[p]).wait()

        def wait_gath(p):
            pltpu.make_async_copy(h_hbm.at[pl.ds(0, B)], rows2.at[p],
                                  gsem[p]).wait()

        def mult_and_scatter(p):
            rowsp = rows2.at[p]

            def jbody(j2, _):
                w16 = wbuf2[p, pl.ds(j2 * L, L)]
                dv16 = dstsc2[p, pl.ds(j2 * L, L)]
                for jj in range(L):
                    jvc = jnp.full((L,), jj, dtype=jnp.int32)
                    wsv = lax.gather(
                        w16, jvc[:, None], _GDN, (1,),
                        mode=lax.GatherScatterMode.PROMISE_IN_BOUNDS)
                    djv = lax.gather(
                        dv16, jvc[:, None], _GDN, (1,),
                        mode=lax.GatherScatterMode.PROMISE_IN_BOUNDS)
                    plsc.addupdate_scatter(den_vm, [djv], wsv, mask=m0)
                    jr = j2 * L + jj
                    for kk in range(Dh // L):
                        rowsp[jr, pl.ds(kk * L, L)] = (
                            rowsp[jr, pl.ds(kk * L, L)] * wsv)
                return _
            lax.fori_loop(0, B // L, jbody, None)
            pltpu.async_copy(rowsp, acc_sh.at[dstsc2.at[p]], ssem[p], add=True)

        def hbody(h, _):
            issue_idx(0, 0)
            pltpu.sync_copy(atab_hbm.at[pl.ds(h * 2 * Nt, 2 * Nt)], a_vm)

            def zden(i, _):
                den_vm[pl.ds(i * L, L)] = zv
                return _
            lax.fori_loop(0, Nt // L, zden, None)

            def zrow(j, _):
                for kk in range(Dh // L):
                    rows2[0, j, pl.ds(kk * L, L)] = zv
                return _
            lax.fori_loop(0, B, zrow, None)

            def zacc(kk, _):
                pltpu.sync_copy(rows2.at[0],
                                acc_sh.at[pl.ds(s * rpt + kk * B, B)])
                return _
            lax.fori_loop(0, rpt // B, zacc, None)
            rem = rpt % B
            if rem:
                pltpu.sync_copy(
                    rows2.at[0].at[pl.ds(0, rem)],
                    acc_sh.at[pl.ds(s * rpt + (rpt // B) * B, rem)])
            plsc.subcore_barrier()

            hoff = h * Nt

            def compute_w(g, p):
                for j2 in range(B // L):
                    sv = srcb2[p, pl.ds(j2 * L, L)]
                    dv = dstb2[p, pl.ds(j2 * L, L)]
                    asr = plsc.load_gather(a_vm, [sv * 2])
                    ads = plsc.load_gather(a_vm, [dv * 2 + 1])
                    w = jnp.exp(_lrelu(asr + ads))
                    wbuf2[p, pl.ds(j2 * L, L)] = w
                    srcadj2[p, pl.ds(j2 * L, L)] = sv + hoff
                    dstsc2[p, pl.ds(j2 * L, L)] = dv

            def stage(g, p, q):
                wait_idx(p)

                @pl.when(g >= 2)
                def _():
                    wait_scat(p)
                compute_w(g, p)
                pltpu.async_copy(h_hbm.at[srcadj2.at[p]], rows2.at[p], gsem[p])

                @pl.when(g + 1 < nb)
                def _():
                    issue_idx(g + 1, q)

                @pl.when(g >= 1)
                def _():
                    wait_gath(q)
                    mult_and_scatter(q)

            def pair(t, _):
                stage(2 * t, 0, 1)
                stage(2 * t + 1, 1, 0)
                return _
            lax.fori_loop(0, nb // 2, pair, None)
            wait_gath(1)
            mult_and_scatter(1)
            wait_scat(0)
            wait_scat(1)
            plsc.subcore_barrier()

            def orow(kk, _):
                r0 = s * rpt + kk * B
                pltpu.sync_copy(acc_sh.at[pl.ds(r0, B)],
                                acc_hbm.at[pl.ds((c * H + h) * Nt + r0, B)])
                return _
            lax.fori_loop(0, rpt // B, orow, None)
            rem2 = rpt % B
            if rem2:
                r0 = s * rpt + (rpt // B) * B
                pltpu.sync_copy(
                    acc_sh.at[pl.ds(r0, rem2)],
                    acc_hbm.at[pl.ds((c * H + h) * Nt + r0, rem2)])
            pltpu.sync_copy(den_vm, den_hbm.at[pl.ds((wid * H + h) * Nt, Nt)])
            plsc.subcore_barrier()
            return _
        lax.fori_loop(0, H, hbody, None)

    return k(h_flat, atab, srcp, dstp)


def _sc_decode(U, V, wp2, srcl, dstl, ELp, Dh):
    B = BD
    elw = ELp // NW
    nb = elw // B
    mesh = plsc.VectorSubcoreMesh(core_axis_name="c", subcore_axis_name="s",
                                  num_cores=NC, num_subcores=NS)

    @functools.partial(
        pl.kernel, mesh=mesh,
        compiler_params=pltpu.CompilerParams(needs_layout_passes=False),
        out_type=jax.ShapeDtypeStruct((ELp, L), _F32),
        scratch_types=[
            pltpu.VMEM((Dh,), _F32),
            pltpu.VMEM((2, B, Dh), _F32),
            pltpu.VMEM((2, B, L), _F32),
            pltpu.VMEM((2, B), jnp.int32),
            pltpu.VMEM((2, B), jnp.int32),
            pltpu.SemaphoreType.DMA,
            pltpu.SemaphoreType.DMA,
            pltpu.SemaphoreType.DMA,
            pltpu.SemaphoreType.DMA,
            pltpu.SemaphoreType.DMA,
            pltpu.SemaphoreType.DMA,
            pltpu.SemaphoreType.DMA,
            pltpu.SemaphoreType.DMA,
        ])
    def k(u_hbm, v_hbm, w_hbm, src_hbm, dst_hbm, p_hbm,
          wp2_vm, ub2, pb2, srcb2, dstb2,
          isem0, isem1, usem0, usem1, vsem0, vsem1, psem0, psem1):
        c = lax.axis_index("c")
        s = lax.axis_index("s")
        wid = s * NC + c
        isem = (isem0, isem1)
        usem = (usem0, usem1)
        vsem = (vsem0, vsem1)
        psem = (psem0, psem1)
        base = wid * elw
        pltpu.sync_copy(w_hbm, wp2_vm)

        def issue_idx(g, p):
            eo = base + g * B
            pltpu.async_copy(src_hbm.at[pl.ds(eo, B)], srcb2.at[p], isem[p])
            pltpu.async_copy(dst_hbm.at[pl.ds(eo, B)], dstb2.at[p], isem[p])

        def wait_idx(p):
            pltpu.make_async_copy(src_hbm.at[pl.ds(0, B)], srcb2.at[p],
                                  isem[p]).wait()
            pltpu.make_async_copy(dst_hbm.at[pl.ds(0, B)], dstb2.at[p],
                                  isem[p]).wait()

        def wait_u(p):
            pltpu.make_async_copy(u_hbm.at[pl.ds(0, B)], ub2.at[p],
                                  usem[p]).wait()

        def wait_v(p):
            pltpu.make_async_copy(v_hbm.at[pl.ds(0, B)], ub2.at[p],
                                  vsem[p]).wait()

        def wait_pst(p):
            pltpu.make_async_copy(pb2.at[p], p_hbm.at[pl.ds(0, B)],
                                  psem[p]).wait()

        def compute_store(g, p):
            def jbody(j, _):
                acc = jnp.zeros((L,), _F32)
                for kk in range(Dh // L):
                    t = ub2[p, j, pl.ds(kk * L, L)]
                    acc = acc + jnp.maximum(t, 0.0) * wp2_vm[pl.ds(kk * L, L)]
                pb2[p, j, pl.ds(0, L)] = acc
                return _
            lax.fori_loop(0, B, jbody, None, unroll=2)
            pltpu.async_copy(pb2.at[p], p_hbm.at[pl.ds(base + g * B, B)],
                             psem[p])

        def stage(g, p, q):
            wait_idx(p)
            pltpu.async_copy(u_hbm.at[srcb2.at[p]], ub2.at[p], usem[p])

            @pl.when(g >= 1)
            def _():
                wait_v(q)

            @pl.when(g + 1 < nb)
            def _():
                issue_idx(g + 1, q)

            @pl.when(g >= 3)
            def _():
                wait_pst(q)

            @pl.when(g >= 1)
            def _():
                compute_store(g - 1, q)
            wait_u(p)
            pltpu.async_copy(v_hbm.at[dstb2.at[p]], ub2.at[p], vsem[p],
                             add=True)

        issue_idx(0, 0)

        def pair(t, _):
            stage(2 * t, 0, 1)
            stage(2 * t + 1, 1, 0)
            return _
        lax.fori_loop(0, nb // 2, pair, None)
        wait_v(1)
        wait_pst(1)
        compute_store(nb - 1, 1)
        wait_pst(0)
        wait_pst(1)

    return k(U, V, wp2, srcl, dstl)


def kernel(x, edge_index, edge_label_index, W1, a_src1, a_dst1, b1,
           W2, a_src2, a_dst2, b2, Wp1, bp1, Wp2, bp2):
    N, Din = x.shape
    H, Dh = a_src1.shape
    E = edge_index.shape[1]
    EL = edge_label_index.shape[1]
    Nt = _rup(N + 64, NS * BD)
    Ep = _rup(E, NW * BA * 2)
    ELp = _rup(EL, NW * BD * 2)

    xp = jnp.pad(x, ((0, Nt - N), (0, 0)))
    pad1 = (N + (jnp.arange(Ep - E, dtype=jnp.int32) % 64)) if Ep > E else None
    srcp = jnp.concatenate([edge_index[0], pad1]) if Ep > E else edge_index[0]
    dstp = jnp.concatenate([edge_index[1], pad1]) if Ep > E else edge_index[1]
    padl = (N + (jnp.arange(ELp - EL, dtype=jnp.int32) % 64)) if ELp > EL else None
    srcl = jnp.concatenate([edge_label_index[0], padl]) if ELp > EL else edge_label_index[0]
    dstl = jnp.concatenate([edge_label_index[1], padl]) if ELp > EL else edge_label_index[1]

    a1w = jnp.stack([a_src1, a_dst1], axis=-1)
    a2w = jnp.concatenate([a_src2, a_dst2], axis=0).T
    b1r = b1.reshape(H, 1, Dh)
    b2r = b2.reshape(1, Dh)
    bp1r = bp1.reshape(1, Dh)
    W2h = W2.astype(jnp.bfloat16)
    W2l = (W2 - W2h.astype(_F32)).astype(jnp.bfloat16)
    W5 = jnp.concatenate([Wp1[:Dh], Wp1[Dh:]], axis=1)
    W5h = W5.astype(jnp.bfloat16)
    W5l = (W5 - W5h.astype(_F32)).astype(jnp.bfloat16)
    wp2f = Wp2.reshape(Dh)
    S7 = jnp.repeat(jnp.eye(8, dtype=_F32), L, axis=0)
    bb7 = jnp.broadcast_to(bp2.reshape(1, 1), (1, 8))

    xh = xp.astype(jnp.bfloat16)
    xl = (xp - xh.astype(_F32)).astype(jnp.bfloat16)
    W1h = W1.astype(jnp.bfloat16)
    W1l = (W1 - W1h.astype(_F32)).astype(jnp.bfloat16)
    h1_3, a1_3 = _tc_encode1(xh, xl, W1h, W1l, a1w, Nt, H, Dh)
    a1m = a1_3.at[:, N:, :].set(-1e30)
    acc1, den1 = _sc_agg(h1_3.reshape(H * Nt, Dh), a1m.reshape(H * 2 * Nt),
                         srcp, dstp, H, Nt, Dh, Ep)
    den1t = den1.reshape(NW, H, Nt).transpose(1, 2, 0)
    h2, a2_ = _tc_combine1(acc1.reshape(NC * H, Nt, Dh), den1t, h1_3, a1m,
                           W2h, W2l, b1r, a2w, Nt, H, Dh)

    a2m = a2_.at[N:].set(-1e30)
    acc2, den2 = _sc_agg(h2, a2m.reshape(2 * Nt), srcp, dstp, 1, Nt, Dh, Ep)
    den2t = den2.reshape(NW, Nt).transpose(1, 0)
    U, V = _tc_combine2(acc2.reshape(NC, Nt, Dh), den2t, h2, a2m,
                        W5h, W5l, b2r, bp1r, Nt, Dh)

    P = _sc_decode(U, V, wp2f, srcl, dstl, ELp, Dh)
    out8 = _tc_finish(P.reshape(ELp // 8, 8 * L), S7, bb7)
    return out8.reshape(ELp)[:EL]


# speedup vs baseline: 1.0606x; 1.0606x over previous
import functools

import jax
import jax.numpy as jnp
from jax import lax
from jax.experimental import pallas as pl
from jax.experimental.pallas import tpu as pltpu
from jax.experimental.pallas import tpu_sc as plsc

NC, NS, L = 2, 16, 16
NW = NC * NS
BA = 64
BD = 128

_F32 = jnp.float32
_HI = jax.lax.Precision.HIGHEST
_GDN = lax.GatherDimensionNumbers(offset_dims=(), collapsed_slice_dims=(0,),
                                  start_index_map=(0,))


def _rup(a, b):
    return (a + b - 1) // b * b


def _lrelu(x):
    return jnp.where(x >= 0, x, 0.2 * x)


def _dot3(a, b):
    ah = a.astype(jnp.bfloat16)
    al = (a - ah.astype(_F32)).astype(jnp.bfloat16)
    bh = b.astype(jnp.bfloat16)
    bl = (b - bh.astype(_F32)).astype(jnp.bfloat16)
    d = jnp.dot(ah, bh, preferred_element_type=_F32)
    d = d + jnp.dot(ah, bl, preferred_element_type=_F32)
    d = d + jnp.dot(al, bh, preferred_element_type=_F32)
    return d


def _tc_encode1(xh, xl, W1h, W1l, a1w, Nt, H, Dh, R=512):
    Din = xh.shape[1]

    def body(xh_ref, xl_ref, wh_ref, wl_ref, aw_ref, h_ref, a_ref):
        xhb, xlb = xh_ref[...], xl_ref[...]
        whb, wlb = wh_ref[...], wl_ref[...]
        hb = jnp.dot(xhb, whb, preferred_element_type=_F32)
        hb = hb + jnp.dot(xhb, wlb, preferred_element_type=_F32)
        hb = hb + jnp.dot(xlb, whb, preferred_element_type=_F32)
        for hh in range(H):
            sl = hb[:, hh * Dh:(hh + 1) * Dh]
            h_ref[hh] = sl
            a_ref[hh] = jnp.dot(sl, aw_ref[hh], preferred_element_type=_F32,
                                precision=_HI)

    return pl.pallas_call(
        body,
        grid=(Nt // R,),
        in_specs=[
            pl.BlockSpec((R, Din), lambda i: (i, 0)),
            pl.BlockSpec((R, Din), lambda i: (i, 0)),
            pl.BlockSpec((Din, H * Dh), lambda i: (0, 0)),
            pl.BlockSpec((Din, H * Dh), lambda i: (0, 0)),
            pl.BlockSpec((H, Dh, 2), lambda i: (0, 0, 0)),
        ],
        out_specs=[
            pl.BlockSpec((H, R, Dh), lambda i: (0, i, 0)),
            pl.BlockSpec((H, R, 2), lambda i: (0, i, 0)),
        ],
        out_shape=[
            jax.ShapeDtypeStruct((H, Nt, Dh), _F32),
            jax.ShapeDtypeStruct((H, Nt, 2), _F32),
        ],
    )(xh, xl, W1h, W1l, a1w)


def _tc_combine1(acc1, den1t, h1_3, a1m, W2h, W2l, b1r, a2w, Nt, H, Dh, R=512):
    def body(acc_ref, den_ref, h1_ref, a1_ref, wh_ref, wl_ref, b1_ref, aw_ref,
             h2_ref, a2_ref):
        es = []
        for h in range(H):
            a1b = a1_ref[h]
            s = a1b[:, 0:1] + a1b[:, 1:2]
            wself = jnp.exp(_lrelu(s))
            num = acc_ref[h] + acc_ref[H + h] + wself * h1_ref[h]
            den = jnp.sum(den_ref[h], axis=1, keepdims=True) + wself
            o = num / (den + 1e-16) + b1_ref[h]
            es.append(jnp.where(o > 0, o, jnp.exp(o) - 1.0))
        e = jnp.concatenate(es, axis=1)
        eh = e.astype(jnp.bfloat16)
        el = (e - eh.astype(_F32)).astype(jnp.bfloat16)
        whb, wlb = wh_ref[...], wl_ref[...]
        h2 = jnp.dot(eh, whb, preferred_element_type=_F32)
        h2 = h2 + jnp.dot(eh, wlb, preferred_element_type=_F32)
        h2 = h2 + jnp.dot(el, whb, preferred_element_type=_F32)
        h2_ref[...] = h2
        a2_ref[...] = jnp.dot(h2, aw_ref[...], preferred_element_type=_F32,
                              precision=_HI)

    return pl.pallas_call(
        body,
        grid=(Nt // R,),
        in_specs=[
            pl.BlockSpec((NC * H, R, Dh), lambda i: (0, i, 0)),
            pl.BlockSpec((H, R, NW), lambda i: (0, i, 0)),
            pl.BlockSpec((H, R, Dh), lambda i: (0, i, 0)),
            pl.BlockSpec((H, R, 2), lambda i: (0, i, 0)),
            pl.BlockSpec((H * Dh, Dh), lambda i: (0, 0)),
            pl.BlockSpec((H * Dh, Dh), lambda i: (0, 0)),
            pl.BlockSpec((H, 1, Dh), lambda i: (0, 0, 0)),
            pl.BlockSpec((Dh, 2), lambda i: (0, 0)),
        ],
        out_specs=[
            pl.BlockSpec((R, Dh), lambda i: (i, 0)),
            pl.BlockSpec((R, 2), lambda i: (i, 0)),
        ],
        out_shape=[
            jax.ShapeDtypeStruct((Nt, Dh), _F32),
            jax.ShapeDtypeStruct((Nt, 2), _F32),
        ],
    )(acc1, den1t, h1_3, a1m, W2h, W2l, b1r, a2w)


def _tc_combine2(acc2, den2t, h2, a2m, W5h, W5l, b2r, bp1r, Nt, Dh, R=512):
    def body(acc_ref, den_ref, h2_ref, a2_ref, wh_ref, wl_ref, b2_ref,
             bp1_ref, u_ref, v_ref):
        a2b = a2_ref[...]
        s = a2b[:, 0:1] + a2b[:, 1:2]
        wself = jnp.exp(_lrelu(s))
        num = acc_ref[0] + acc_ref[1] + wself * h2_ref[...]
        den = jnp.sum(den_ref[...], axis=1, keepdims=True) + wself
        z = num / (den + 1e-16) + b2_ref[...]
        zh = z.astype(jnp.bfloat16)
        zl = (z - zh.astype(_F32)).astype(jnp.bfloat16)
        whb, wlb = wh_ref[...], wl_ref[...]
        uv = jnp.dot(zh, whb, preferred_element_type=_F32)
        uv = uv + jnp.dot(zh, wlb, preferred_element_type=_F32)
        uv = uv + jnp.dot(zl, whb, preferred_element_type=_F32)
        u_ref[...] = uv[:, :Dh] + bp1_ref[...]
        v_ref[...] = uv[:, Dh:]

    return pl.pallas_call(
        body,
        grid=(Nt // R,),
        in_specs=[
            pl.BlockSpec((NC, R, Dh), lambda i: (0, i, 0)),
            pl.BlockSpec((R, NW), lambda i: (i, 0)),
            pl.BlockSpec((R, Dh), lambda i: (i, 0)),
            pl.BlockSpec((R, 2), lambda i: (i, 0)),
            pl.BlockSpec((Dh, 2 * Dh), lambda i: (0, 0)),
            pl.BlockSpec((Dh, 2 * Dh), lambda i: (0, 0)),
            pl.BlockSpec((1, Dh), lambda i: (0, 0)),
            pl.BlockSpec((1, Dh), lambda i: (0, 0)),
        ],
        out_specs=[
            pl.BlockSpec((R, Dh), lambda i: (i, 0)),
            pl.BlockSpec((R, Dh), lambda i: (i, 0)),
        ],
        out_shape=[
            jax.ShapeDtypeStruct((Nt, Dh), _F32),
            jax.ShapeDtypeStruct((Nt, Dh), _F32),
        ],
    )(acc2, den2t, h2, a2m, W5h, W5l, b2r, bp1r)


def _tc_finish(P2, S, bb, R=1024):
    M = P2.shape[0]

    def body(p_ref, s_ref, b_ref, o_ref):
        o_ref[...] = jnp.dot(p_ref[...], s_ref[...], preferred_element_type=_F32,
                             precision=_HI) + b_ref[...]

    return pl.pallas_call(
        body,
        grid=(M // R,),
        in_specs=[
            pl.BlockSpec((R, 128), lambda i: (i, 0)),
            pl.BlockSpec((128, 8), lambda i: (0, 0)),
            pl.BlockSpec((1, 8), lambda i: (0, 0)),
        ],
        out_specs=pl.BlockSpec((R, 8), lambda i: (i, 0)),
        out_shape=jax.ShapeDtypeStruct((M, 8), _F32),
    )(P2, S, bb)


def _sc_agg(h_flat, atab, srcp, dstp, H, Nt, Dh, Ep):
    B = BA
    epw = Ep // NW
    nb = epw // B
    rpt = Nt // NS
    nzc = rpt // B
    mesh = plsc.VectorSubcoreMesh(core_axis_name="c", subcore_axis_name="s",
                                  num_cores=NC, num_subcores=NS)

    @functools.partial(
        pl.kernel, mesh=mesh,
        compiler_params=pltpu.CompilerParams(needs_layout_passes=False,
                                             internal_scratch_in_bytes=2048),
        out_type=[
            jax.ShapeDtypeStruct((NC * H * Nt, Dh), _F32),
            jax.ShapeDtypeStruct((NW * H * Nt,), _F32),
        ],
        scratch_types=[
            pltpu.VMEM((2 * Nt,), _F32),
            pltpu.VMEM((Nt,), _F32),
            pltpu.VMEM((2, B, Dh), _F32),
            pltpu.VMEM((2, B), jnp.int32),
            pltpu.VMEM((2, B), jnp.int32),
            pltpu.VMEM((2, B), jnp.int32),
            pltpu.VMEM((2, B), jnp.int32),
            pltpu.VMEM((2, B), _F32),
            pltpu.VMEM_SHARED((Nt, Dh), _F32),
            pltpu.SemaphoreType.DMA,
            pltpu.SemaphoreType.DMA,
            pltpu.SemaphoreType.DMA,
            pltpu.SemaphoreType.DMA,
            pltpu.SemaphoreType.DMA,
            pltpu.SemaphoreType.DMA,
        ])
    def k(h_hbm, atab_hbm, src_hbm, dst_hbm, acc_hbm, den_hbm,
          a_vm, den_vm, rows2, srcb2, dstb2, srcadj2, dstsc2, wbuf2,
          acc_sh, isem0, isem1, gsem0, gsem1, ssem0, ssem1):
        c = lax.axis_index("c")
        s = lax.axis_index("s")
        wid = s * NC + c
        zv = jnp.zeros((L,), _F32)
        m0 = lax.iota(jnp.int32, L) == 0
        isem = (isem0, isem1)
        gsem = (gsem0, gsem1)
        ssem = (ssem0, ssem1)
        base = wid * epw

        def issue_idx(g, p):
            eo = base + g * B
            pltpu.async_copy(src_hbm.at[pl.ds(eo, B)], srcb2.at[p], isem[p])
            pltpu.async_copy(dst_hbm.at[pl.ds(eo, B)], dstb2.at[p], isem[p])

        def wait_idx(p):
            pltpu.make_async_copy(src_hbm.at[pl.ds(0, B)], srcb2.at[p],
                                  isem[p]).wait()
            pltpu.make_async_copy(dst_hbm.at[pl.ds(0, B)], dstb2.at[p],
                                  isem[p]).wait()

        def wait_scat(p):
            pltpu.make_async_copy(rows2.at[p], acc_sh.at[pl.ds(0, B)],
                                  ssem[p]).wait()

        def wait_gath(p):
            pltpu.make_async_copy(h_hbm.at[pl.ds(0, B)], rows2.at[p],
                                  gsem[p]).wait()

        def mult_and_scatter(p):
            rowsp = rows2.at[p]

            def jbody(j2, _):
                w16 = wbuf2[p, pl.ds(j2 * L, L)]
                dv16 = dstsc2[p, pl.ds(j2 * L, L)]
                for jj in range(L):
                    jvc = jnp.full((L,), jj, dtype=jnp.int32)
                    wsv = lax.gather(
                        w16, jvc[:, None], _GDN, (1,),
                        mode=lax.GatherScatterMode.PROMISE_IN_BOUNDS)
                    djv = lax.gather(
                        dv16, jvc[:, None], _GDN, (1,),
                        mode=lax.GatherScatterMode.PROMISE_IN_BOUNDS)
                    plsc.addupdate_scatter(den_vm, [djv], wsv, mask=m0)
                    jr = j2 * L + jj
                    for kk in range(Dh // L):
                        rowsp[jr, pl.ds(kk * L, L)] = (
                            rowsp[jr, pl.ds(kk * L, L)] * wsv)
                return _
            lax.fori_loop(0, B // L, jbody, None)
            pltpu.async_copy(rowsp, acc_sh.at[dstsc2.at[p]], ssem[p], add=True)

        def hbody(h, _):
            issue_idx(0, 0)
            pltpu.sync_copy(atab_hbm.at[pl.ds(h * 2 * Nt, 2 * Nt)], a_vm)

            def zden(i, _):
                den_vm[pl.ds(i * L, L)] = zv
                return _
            lax.fori_loop(0, Nt // L, zden, None)

            def zrow(j, _):
                for kk in range(Dh // L):
                    rows2[0, j, pl.ds(kk * L, L)] = zv
                return _
            lax.fori_loop(0, B, zrow, None)

            def zacc(kk, _):
                pltpu.sync_copy(rows2.at[0],
                                acc_sh.at[pl.ds(s * rpt + kk * B, B)])
                return _
            lax.fori_loop(0, rpt // B, zacc, None)
            rem = rpt % B
            if rem:
                pltpu.sync_copy(
                    rows2.at[0].at[pl.ds(0, rem)],
                    acc_sh.at[pl.ds(s * rpt + (rpt // B) * B, rem)])
            plsc.subcore_barrier()

            hoff = h * Nt

            def compute_w(g, p):
                for j2 in range(B // L):
                    sv = srcb2[p, pl.ds(j2 * L, L)]
                    dv = dstb2[p, pl.ds(j2 * L, L)]
                    asr = plsc.load_gather(a_vm, [sv * 2])
                    ads = plsc.load_gather(a_vm, [dv * 2 + 1])
                    w = jnp.exp(_lrelu(asr + ads))
                    wbuf2[p, pl.ds(j2 * L, L)] = w
                    srcadj2[p, pl.ds(j2 * L, L)] = sv + hoff
                    dstsc2[p, pl.ds(j2 * L, L)] = dv

            def stage(g, p, q):
                wait_idx(p)

                @pl.when(g >= 2)
                def _():
                    wait_scat(p)
                compute_w(g, p)
                pltpu.async_copy(h_hbm.at[srcadj2.at[p]], rows2.at[p], gsem[p])

                @pl.when(g + 1 < nb)
                def _():
                    issue_idx(g + 1, q)

                @pl.when(g >= 1)
                def _():
                    wait_gath(q)
                    mult_and_scatter(q)

            def pair(t, _):
                stage(2 * t, 0, 1)
                stage(2 * t + 1, 1, 0)
                return _
            lax.fori_loop(0, nb // 2, pair, None)
            wait_gath(1)
            mult_and_scatter(1)
            wait_scat(0)
            wait_scat(1)
            plsc.subcore_barrier()

            def orow(kk, _):
                r0 = s * rpt + kk * B
                pltpu.sync_copy(acc_sh.at[pl.ds(r0, B)],
                                acc_hbm.at[pl.ds((c * H + h) * Nt + r0, B)])
                return _
            lax.fori_loop(0, rpt // B, orow, None)
            rem2 = rpt % B
            if rem2:
                r0 = s * rpt + (rpt // B) * B
                pltpu.sync_copy(
                    acc_sh.at[pl.ds(r0, rem2)],
                    acc_hbm.at[pl.ds((c * H + h) * Nt + r0, rem2)])
            pltpu.sync_copy(den_vm, den_hbm.at[pl.ds((wid * H + h) * Nt, Nt)])
            plsc.subcore_barrier()
            return _
        lax.fori_loop(0, H, hbody, None)

    return k(h_flat, atab, srcp, dstp)


def _sc_decode(U, V, wp2, srcl, dstl, ELp, Dh):
    B = BD
    elw = ELp // NW
    nb = elw // B
    mesh = plsc.VectorSubcoreMesh(core_axis_name="c", subcore_axis_name="s",
                                  num_cores=NC, num_subcores=NS)

    @functools.partial(
        pl.kernel, mesh=mesh,
        compiler_params=pltpu.CompilerParams(needs_layout_passes=False),
        out_type=jax.ShapeDtypeStruct((ELp, L), _F32),
        scratch_types=[
            pltpu.VMEM((Dh,), _F32),
            pltpu.VMEM((2, B, Dh), _F32),
            pltpu.VMEM((2, B, Dh), _F32),
            pltpu.VMEM((2, B, L), _F32),
            pltpu.VMEM((2, B), jnp.int32),
            pltpu.VMEM((2, B), jnp.int32),
            pltpu.SemaphoreType.DMA,
            pltpu.SemaphoreType.DMA,
            pltpu.SemaphoreType.DMA,
            pltpu.SemaphoreType.DMA,
            pltpu.SemaphoreType.DMA,
            pltpu.SemaphoreType.DMA,
            pltpu.SemaphoreType.DMA,
            pltpu.SemaphoreType.DMA,
        ])
    def k(u_hbm, v_hbm, w_hbm, src_hbm, dst_hbm, p_hbm,
          wp2_vm, ub2, vb2, pb2, srcb2, dstb2,
          isem0, isem1, usem0, usem1, vsem0, vsem1, psem0, psem1):
        c = lax.axis_index("c")
        s = lax.axis_index("s")
        wid = s * NC + c
        isem = (isem0, isem1)
        usem = (usem0, usem1)
        vsem = (vsem0, vsem1)
        psem = (psem0, psem1)
        base = wid * elw
        pltpu.sync_copy(w_hbm, wp2_vm)

        def issue_idx(g, p):
            eo = base + g * B
            pltpu.async_copy(src_hbm.at[pl.ds(eo, B)], srcb2.at[p], isem[p])
            pltpu.async_copy(dst_hbm.at[pl.ds(eo, B)], dstb2.at[p], isem[p])

        def wait_idx(p):
            pltpu.make_async_copy(src_hbm.at[pl.ds(0, B)], srcb2.at[p],
                                  isem[p]).wait()
            pltpu.make_async_copy(dst_hbm.at[pl.ds(0, B)], dstb2.at[p],
                                  isem[p]).wait()

        def wait_rows(p):
            pltpu.make_async_copy(u_hbm.at[pl.ds(0, B)], ub2.at[p],
                                  usem[p]).wait()
            pltpu.make_async_copy(v_hbm.at[pl.ds(0, B)], vb2.at[p],
                                  vsem[p]).wait()

        def wait_pst(p):
            pltpu.make_async_copy(pb2.at[p], p_hbm.at[pl.ds(0, B)],
                                  psem[p]).wait()

        def compute_store(g, p):
            def jbody(j, _):
                acc = jnp.zeros((L,), _F32)
                for kk in range(Dh // L):
                    t = ub2[p, j, pl.ds(kk * L, L)] + vb2[p, j, pl.ds(kk * L, L)]
                    acc = acc + jnp.maximum(t, 0.0) * wp2_vm[pl.ds(kk * L, L)]
                pb2[p, j, pl.ds(0, L)] = acc
                return _
            lax.fori_loop(0, B, jbody, None, unroll=2)
            pltpu.async_copy(pb2.at[p], p_hbm.at[pl.ds(base + g * B, B)],
                             psem[p])

        def stage(g, p, q):
            wait_idx(p)
            pltpu.async_copy(u_hbm.at[srcb2.at[p]], ub2.at[p], usem[p])
            pltpu.async_copy(v_hbm.at[dstb2.at[p]], vb2.at[p], vsem[p])

            @pl.when(g >= 1)
            def _():
                wait_rows(q)

            @pl.when(g + 1 < nb)
            def _():
                issue_idx(g + 1, q)

            @pl.when(g >= 3)
            def _():
                wait_pst(q)

            @pl.when(g >= 1)
            def _():
                compute_store(g - 1, q)

        issue_idx(0, 0)

        def pair(t, _):
            stage(2 * t, 0, 1)
            stage(2 * t + 1, 1, 0)
            return _
        lax.fori_loop(0, nb // 2, pair, None)
        wait_rows(1)
        wait_pst(1)
        compute_store(nb - 1, 1)
        wait_pst(0)
        wait_pst(1)

    return k(U, V, wp2, srcl, dstl)


def kernel(x, edge_index, edge_label_index, W1, a_src1, a_dst1, b1,
           W2, a_src2, a_dst2, b2, Wp1, bp1, Wp2, bp2):
    N, Din = x.shape
    H, Dh = a_src1.shape
    E = edge_index.shape[1]
    EL = edge_label_index.shape[1]
    Nt = _rup(N + 64, NS * BD)
    Ep = _rup(E, NW * BA * 2)
    ELp = _rup(EL, NW * BD * 2)

    xp = jnp.pad(x, ((0, Nt - N), (0, 0)))
    pad1 = (N + (jnp.arange(Ep - E, dtype=jnp.int32) % 64)) if Ep > E else None
    srcp = jnp.concatenate([edge_index[0], pad1]) if Ep > E else edge_index[0]
    dstp = jnp.concatenate([edge_index[1], pad1]) if Ep > E else edge_index[1]
    padl = (N + (jnp.arange(ELp - EL, dtype=jnp.int32) % 64)) if ELp > EL else None
    srcl = jnp.concatenate([edge_label_index[0], padl]) if ELp > EL else edge_label_index[0]
    dstl = jnp.concatenate([edge_label_index[1], padl]) if ELp > EL else edge_label_index[1]

    a1w = jnp.stack([a_src1, a_dst1], axis=-1)
    a2w = jnp.concatenate([a_src2, a_dst2], axis=0).T
    b1r = b1.reshape(H, 1, Dh)
    b2r = b2.reshape(1, Dh)
    bp1r = bp1.reshape(1, Dh)
    W2h = W2.astype(jnp.bfloat16)
    W2l = (W2 - W2h.astype(_F32)).astype(jnp.bfloat16)
    W5 = jnp.concatenate([Wp1[:Dh], Wp1[Dh:]], axis=1)
    W5h = W5.astype(jnp.bfloat16)
    W5l = (W5 - W5h.astype(_F32)).astype(jnp.bfloat16)
    wp2f = Wp2.reshape(Dh)
    S7 = jnp.repeat(jnp.eye(8, dtype=_F32), L, axis=0)
    bb7 = jnp.broadcast_to(bp2.reshape(1, 1), (1, 8))

    xh = xp.astype(jnp.bfloat16)
    xl = (xp - xh.astype(_F32)).astype(jnp.bfloat16)
    W1h = W1.astype(jnp.bfloat16)
    W1l = (W1 - W1h.astype(_F32)).astype(jnp.bfloat16)
    h1_3, a1_3 = _tc_encode1(xh, xl, W1h, W1l, a1w, Nt, H, Dh)
    a1m = a1_3.at[:, N:, :].set(-1e30)
    acc1, den1 = _sc_agg(h1_3.reshape(H * Nt, Dh), a1m.reshape(H * 2 * Nt),
                         srcp, dstp, H, Nt, Dh, Ep)
    den1t = den1.reshape(NW, H, Nt).transpose(1, 2, 0)
    h2, a2_ = _tc_combine1(acc1.reshape(NC * H, Nt, Dh), den1t, h1_3, a1m,
                           W2h, W2l, b1r, a2w, Nt, H, Dh)

    a2m = a2_.at[N:].set(-1e30)
    acc2, den2 = _sc_agg(h2, a2m.reshape(2 * Nt), srcp, dstp, 1, Nt, Dh, Ep)
    den2t = den2.reshape(NW, Nt).transpose(1, 0)
    U, V = _tc_combine2(acc2.reshape(NC, Nt, Dh), den2t, h2, a2m,
                        W5h, W5l, b2r, bp1r, Nt, Dh)

    P = _sc_decode(U, V, wp2f, srcl, dstl, ELp, Dh)
    out8 = _tc_finish(P.reshape(ELp // 8, 8 * L), S7, bb7)
    return out8.reshape(ELp)[:EL]
